# baseline (speedup 1.0000x reference)
import functools

import jax
import jax.numpy as jnp
from jax.experimental import pallas as pl
from jax.experimental.pallas import tpu as pltpu

_E = 8
_SCALE = 0.01
_BT = 128


def _ffn_body(blk_ref, exp_ref, off_ref,
              x_ref, w1_ref, b1_ref, w2_ref, b2_ref, wp_ref, bp_ref,
              out_ref):
    s = pl.program_id(0)
    n_steps = off_ref[_E + 1]
    blk = blk_ref[s]
    e = exp_ref[s]
    prev = jnp.where(s == 0, 0, s - 1)
    first = jnp.logical_or(s == 0, blk_ref[prev] != blk)
    valid = s < n_steps

    @pl.when(valid)
    def _():
        bt, d = x_ref.shape
        p = blk * bt + jax.lax.broadcasted_iota(jnp.int32, (bt, 1), 0)
        mask = jnp.logical_and(p >= off_ref[e], p < off_ref[e + 1])
        xb = x_ref[...].astype(jnp.bfloat16)
        w1 = w1_ref[0].astype(jnp.bfloat16)
        w2 = w2_ref[0].astype(jnp.bfloat16)
        h1 = jnp.dot(xb, w1, preferred_element_type=jnp.float32)
        h1 = h1 + b1_ref[0]
        h2 = jnp.dot(xb, w2, preferred_element_type=jnp.float32)
        h2 = h2 + b2_ref[0]
        act = h1 * (h2 * jax.nn.sigmoid(h2))
        o = jnp.dot(act.astype(jnp.bfloat16), wp_ref[0].astype(jnp.bfloat16),
                    preferred_element_type=jnp.float32)
        o = o + bp_ref[0]
        contrib = jnp.where(mask, o, 0.0)

        @pl.when(first)
        def _():
            out_ref[...] = contrib

        @pl.when(jnp.logical_not(first))
        def _():
            out_ref[...] = out_ref[...] + contrib


def _grouped_ffn(x_sorted, W1, b1, W2, b2, Wp, bp, blk_ids, exp_ids, off):
    T, D = x_sorted.shape
    H = W1.shape[-1]
    NS = blk_ids.shape[0]
    grid_spec = pltpu.PrefetchScalarGridSpec(
        num_scalar_prefetch=3,
        grid=(NS,),
        in_specs=[
            pl.BlockSpec((_BT, D), lambda s, blk, exp, off: (blk[s], 0)),
            pl.BlockSpec((1, D, H), lambda s, blk, exp, off: (exp[s], 0, 0)),
            pl.BlockSpec((1, 1, H), lambda s, blk, exp, off: (exp[s], 0, 0)),
            pl.BlockSpec((1, D, H), lambda s, blk, exp, off: (exp[s], 0, 0)),
            pl.BlockSpec((1, 1, H), lambda s, blk, exp, off: (exp[s], 0, 0)),
            pl.BlockSpec((1, H, D), lambda s, blk, exp, off: (exp[s], 0, 0)),
            pl.BlockSpec((1, 1, D), lambda s, blk, exp, off: (exp[s], 0, 0)),
        ],
        out_specs=pl.BlockSpec((_BT, D), lambda s, blk, exp, off: (blk[s], 0)),
    )
    return pl.pallas_call(
        _ffn_body,
        grid_spec=grid_spec,
        out_shape=jax.ShapeDtypeStruct((T, D), jnp.float32),
    )(blk_ids, exp_ids, off, x_sorted, W1,
      b1.reshape(b1.shape[0], 1, b1.shape[1]), W2,
      b2.reshape(b2.shape[0], 1, b2.shape[1]), Wp,
      bp.reshape(bp.shape[0], 1, bp.shape[1]))


def kernel(x, gate_W, noise_weight, W1, b1, W2, b2, Wp, bp, noise):
    x_flat = x.reshape(-1, x.shape[-1])
    T, D = x_flat.shape
    E = gate_W.shape[-1]
    logits = x_flat @ gate_W
    logits_noisy = logits + noise * noise_weight[None, :]
    topi_flat = jnp.argmax(logits_noisy, axis=-1).astype(jnp.int32)
    gw_mean = jax.nn.softmax(logits, axis=-1).mean(axis=0)
    lb_loss = jnp.mean((gw_mean - 1.0 / E) ** 2) * _SCALE

    counts = jnp.sum(topi_flat[:, None] == jnp.arange(E)[None, :], axis=0,
                     dtype=jnp.int32)
    offsets = jnp.concatenate(
        [jnp.zeros((1,), jnp.int32), jnp.cumsum(counts)]).astype(jnp.int32)
    sort_idx = jnp.argsort(topi_flat)
    x_sorted = x_flat[sort_idx]

    NB = T // _BT
    NS = NB + E - 1
    b = jnp.arange(NB, dtype=jnp.int32)
    inter = jnp.logical_and(
        offsets[:-1][None, :] < (b[:, None] + 1) * _BT,
        offsets[1:][None, :] > b[:, None] * _BT)
    flat = inter.reshape(-1)
    pos = jnp.cumsum(flat) - 1
    n = flat.sum().astype(jnp.int32)
    bflat = jnp.broadcast_to(b[:, None], (NB, E)).reshape(-1)
    eflat = jnp.broadcast_to(jnp.arange(E, dtype=jnp.int32)[None, :],
                             (NB, E)).reshape(-1)
    dest = jnp.where(flat, pos, NS)
    blk_ids = jnp.zeros((NS,), jnp.int32).at[dest].set(bflat, mode='drop')
    exp_ids = jnp.zeros((NS,), jnp.int32).at[dest].set(eflat, mode='drop')
    tail = jnp.arange(NS) < n
    blk_ids = jnp.where(tail, blk_ids, blk_ids[n - 1])
    exp_ids = jnp.where(tail, exp_ids, exp_ids[n - 1])
    off = jnp.concatenate([offsets, n[None]]).astype(jnp.int32)

    out_sorted = _grouped_ffn(x_sorted, W1, b1, W2, b2, Wp, bp,
                              blk_ids, exp_ids, off)
    final_flat = jnp.zeros((T, D), jnp.float32).at[sort_idx].set(out_sorted)
    final = final_flat.reshape(x.shape)
    return final, topi_flat[:, None], lb_loss


# speedup vs baseline: 1.2662x; 1.2662x over previous
import functools

import jax
import jax.numpy as jnp
from jax.experimental import pallas as pl
from jax.experimental.pallas import tpu as pltpu

_E = 8
_SCALE = 0.01
_BT = 128
_NH = 2


def _ffn_body(off_ref, x_ref, w1_ref, b1_ref, w2_ref, b2_ref, wp_ref, bp_ref,
              out_ref):
    e = pl.program_id(0)
    h = pl.program_id(1)
    lo = off_ref[e]
    hi = off_ref[e + 1]
    astart = (lo // 8) * 8
    nch = pl.cdiv(hi - astart, _BT)
    T = x_ref.shape[0]

    def chunk(i, _):
        ustart = astart + i * _BT
        start = jnp.minimum(ustart, T - _BT)
        xb = x_ref[pl.ds(start, _BT), :]
        h1 = jnp.dot(xb, w1_ref[0], preferred_element_type=jnp.float32)
        h1 = h1 + b1_ref[0]
        h2 = jnp.dot(xb, w2_ref[0], preferred_element_type=jnp.float32)
        h2 = h2 + b2_ref[0]
        act = h1 * (h2 * jax.nn.sigmoid(h2))
        o = jnp.dot(act, wp_ref[0], preferred_element_type=jnp.float32)
        p = start + jax.lax.broadcasted_iota(jnp.int32, (_BT, 1), 0)
        mask = jnp.logical_and(p >= jnp.maximum(lo, ustart), p < hi)
        prev = out_ref[pl.ds(start, _BT), :]

        @pl.when(h == 0)
        def _():
            out_ref[pl.ds(start, _BT), :] = jnp.where(mask, o + bp_ref[0],
                                                      prev)

        @pl.when(h != 0)
        def _():
            out_ref[pl.ds(start, _BT), :] = prev + jnp.where(mask, o, 0.0)

        return 0

    jax.lax.fori_loop(0, nch, chunk, 0)


def _grouped_ffn(x_sorted, W1, b1, W2, b2, Wp, bp, off):
    T, D = x_sorted.shape
    H = W1.shape[-1]
    HC = H // _NH
    grid_spec = pltpu.PrefetchScalarGridSpec(
        num_scalar_prefetch=1,
        grid=(_E, _NH),
        in_specs=[
            pl.BlockSpec((T, D), lambda e, h, off: (0, 0)),
            pl.BlockSpec((1, D, HC), lambda e, h, off: (e, 0, h)),
            pl.BlockSpec((1, 1, HC), lambda e, h, off: (e, 0, h)),
            pl.BlockSpec((1, D, HC), lambda e, h, off: (e, 0, h)),
            pl.BlockSpec((1, 1, HC), lambda e, h, off: (e, 0, h)),
            pl.BlockSpec((1, HC, D), lambda e, h, off: (e, h, 0)),
            pl.BlockSpec((1, 1, D), lambda e, h, off: (e, 0, 0)),
        ],
        out_specs=pl.BlockSpec((T, D), lambda e, h, off: (0, 0)),
    )
    return pl.pallas_call(
        _ffn_body,
        grid_spec=grid_spec,
        out_shape=jax.ShapeDtypeStruct((T, D), jnp.float32),
    )(off, x_sorted, W1,
      b1.reshape(b1.shape[0], 1, b1.shape[1]), W2,
      b2.reshape(b2.shape[0], 1, b2.shape[1]), Wp,
      bp.reshape(bp.shape[0], 1, bp.shape[1]))


def kernel(x, gate_W, noise_weight, W1, b1, W2, b2, Wp, bp, noise):
    x_flat = x.reshape(-1, x.shape[-1])
    T, D = x_flat.shape
    E = gate_W.shape[-1]
    logits = x_flat @ gate_W
    logits_noisy = logits + noise * noise_weight[None, :]
    topi_flat = jnp.argmax(logits_noisy, axis=-1).astype(jnp.int32)
    gw_mean = jax.nn.softmax(logits, axis=-1).mean(axis=0)
    lb_loss = jnp.mean((gw_mean - 1.0 / E) ** 2) * _SCALE

    counts = jnp.sum(topi_flat[:, None] == jnp.arange(E)[None, :], axis=0,
                     dtype=jnp.int32)
    off = jnp.concatenate(
        [jnp.zeros((1,), jnp.int32), jnp.cumsum(counts)]).astype(jnp.int32)
    sort_idx = jnp.argsort(topi_flat)
    x_sorted = x_flat[sort_idx]

    out_sorted = _grouped_ffn(x_sorted, W1, b1, W2, b2, Wp, bp, off)
    final_flat = jnp.zeros((T, D), jnp.float32).at[sort_idx].set(out_sorted)
    final = final_flat.reshape(x.shape)
    return final, topi_flat[:, None], lb_loss
